# baseline (speedup 1.0000x reference)
import jax
import jax.numpy as jnp
from jax import lax
from jax.experimental import pallas as pl
from jax.experimental.pallas import tpu as pltpu
from jax.experimental.pallas import tpu_sc as plsc

B, S, D = 1, 2048, 768
E, K, F = 8, 2, 768
ALPHA, LIMIT = 1.702, 7.0

P = S * K
BR = 256
NB = P // BR + E
NR = NB * BR
L = 16
NC, NSUB = 2, 16
NW = NC * NSUB
PW = P // NSUB
VPW = PW // L
GR = NR // NW
GC = 64
TPW = S // NW
ZV = (NR // NSUB) // L


def _router_body(x_ref, wg_ref, bg_ref, sel_ref, w_ref):
    x = x_ref[...]
    logits = jax.lax.dot_general(
        x, wg_ref[...], (((1,), (1,)), ((), ())),
        preferred_element_type=jnp.float32)
    logits = logits + bg_ref[...]
    idx8 = jax.lax.broadcasted_iota(jnp.int32, (S, E), 1)
    m1 = jnp.max(logits, axis=1, keepdims=True)
    a1 = jnp.min(jnp.where(logits == m1, idx8, E), axis=1, keepdims=True)
    masked = jnp.where(idx8 == a1, -jnp.inf, logits)
    m2 = jnp.max(masked, axis=1, keepdims=True)
    a2 = jnp.min(jnp.where(masked == m2, idx8, E), axis=1, keepdims=True)
    w1 = jax.nn.sigmoid(m1 - m2)
    sel_ref[...] = jnp.concatenate([a1, a2], axis=1)
    w_ref[...] = jnp.concatenate([w1, 1.0 - w1], axis=1)


def _dispatch_body(sel_hbm, wp_hbm, x_hbm, xg_hbm, wmap_hbm, pos_hbm,
                   bexp_hbm, nblk_hbm, keyv, wv, rankv, tokv, posA, posB,
                   cnt_tbl, counts_all, zb_i, zb_f, idxv, wsl, rb0, rb1,
                   bexpv, nblkv, counts_sh, tok_sh, w_sh,
                   gs0, gs1, ws0, ws1):
    c = lax.axis_index("c")
    s = lax.axis_index("s")
    wid = s * NC + c
    pbase = s * PW
    iota = lax.iota(jnp.int32, L)
    zero16 = jnp.zeros((L,), jnp.int32)

    _scopeA = jax.named_scope("phaseA_sort"); _scopeA.__enter__()
    pltpu.sync_copy(sel_hbm.at[pl.ds(pbase, PW)], keyv)
    pltpu.sync_copy(wp_hbm.at[pl.ds(pbase, PW)], wv)
    run16 = zero16
    for j in range(VPW):
        k16 = keyv[pl.ds(j * L, L)]
        rank16 = zero16
        cnt16 = zero16
        for l in range(L):
            kl = k16[l]
            mask_l = jnp.where(iota > l, 1, 0)
            rank16 = rank16 + jnp.where(k16 == kl, 1, 0) * mask_l
            cnt16 = cnt16 + jnp.where(iota == kl, 1, 0)
        prev16 = zero16
        for e in range(E):
            prev16 = prev16 + jnp.where(
                k16 == e, jnp.full((L,), run16[e], jnp.int32), 0)
        rankv[pl.ds(j * L, L)] = prev16 + rank16
        run16 = run16 + cnt16
    cnt_tbl[...] = run16
    pltpu.sync_copy(cnt_tbl, counts_sh.at[pl.ds(s * L, L)])
    plsc.subcore_barrier()
    _scopeA.__exit__(None, None, None)
    _scopeB = jax.named_scope("phaseBC_pos"); _scopeB.__enter__()

    pltpu.sync_copy(counts_sh, counts_all)
    totals = zero16
    prefix = zero16
    for w2 in range(NSUB):
        c16 = counts_all[pl.ds(w2 * L, L)]
        m = -((w2 - s) >> 31)
        prefix = prefix + c16 * jnp.full((L,), m, jnp.int32)
        totals = totals + c16
    padded = ((totals + (BR - 1)) >> 8) << 8
    ends = zero16
    for e in range(E):
        ends = ends + jnp.where(
            iota >= e, jnp.full((L,), padded[e], jnp.int32), 0)
    offs = ends - padded
    base16 = offs + prefix
    base_sc = [base16[e] for e in range(E)]
    nblocks = ends[E - 1] >> 8

    for j in range(VPW):
        k16 = keyv[pl.ds(j * L, L)]
        b16 = zero16
        for e in range(E):
            b16 = b16 + jnp.where(
                k16 == e, jnp.full((L,), base_sc[e], jnp.int32), 0)
        p16 = b16 + rankv[pl.ds(j * L, L)]
        if j < VPW // 2:
            posA[pl.ds(j * L, L)] = p16
        else:
            posB[pl.ds((j - VPW // 2) * L, L)] = p16
        tokv[pl.ds(j * L, L)] = (
            jnp.full((L,), pbase + j * L, jnp.int32) + iota) >> 1

    @pl.when(c == 0)
    def _():
        pltpu.sync_copy(posA, pos_hbm.at[pl.ds(pbase, PW // 2)])
        pltpu.sync_copy(posB, pos_hbm.at[pl.ds(pbase + PW // 2, PW // 2)])

    _scopeB.__exit__(None, None, None)
    _scopeD = jax.named_scope("phaseD_scatter"); _scopeD.__enter__()
    zf16 = jnp.zeros((L,), jnp.float32)
    for j in range(ZV):
        zb_i[pl.ds(j * L, L)] = zero16
        zb_f[pl.ds(j * L, L)] = zf16
    pltpu.sync_copy(zb_i, tok_sh.at[pl.ds(s * (NR // NSUB), NR // NSUB)])
    pltpu.sync_copy(zb_f, w_sh.at[pl.ds(s * (NR // NSUB), NR // NSUB)])
    plsc.subcore_barrier()
    pltpu.sync_copy(tokv.at[pl.ds(0, PW // 2)], tok_sh.at[posA])
    pltpu.sync_copy(tokv.at[pl.ds(PW // 2, PW // 2)], tok_sh.at[posB])
    pltpu.sync_copy(wv.at[pl.ds(0, PW // 2)], w_sh.at[posA])
    pltpu.sync_copy(wv.at[pl.ds(PW // 2, PW // 2)], w_sh.at[posB])
    plsc.subcore_barrier()

    _scopeD.__exit__(None, None, None)
    _scopeE = jax.named_scope("phaseE_gather"); _scopeE.__enter__()
    gbase = wid * GR
    pltpu.sync_copy(tok_sh.at[pl.ds(gbase, GR)], idxv)
    pltpu.sync_copy(w_sh.at[pl.ds(gbase, GR)], wsl)
    pltpu.sync_copy(wsl, wmap_hbm.at[pl.ds(gbase, GR)])
    g0 = pltpu.async_copy(x_hbm.at[idxv.at[pl.ds(0, GC)]], rb0, gs0)
    g0.wait()
    g1 = pltpu.async_copy(x_hbm.at[idxv.at[pl.ds(GC, GC)]], rb1, gs1)
    w0 = pltpu.async_copy(rb0, xg_hbm.at[pl.ds(gbase, GC)], ws0)
    g1.wait()
    w0.wait()
    g2 = pltpu.async_copy(x_hbm.at[idxv.at[pl.ds(2 * GC, GC)]], rb0, gs0)
    w1 = pltpu.async_copy(rb1, xg_hbm.at[pl.ds(gbase + GC, GC)], ws1)
    g2.wait()
    w2 = pltpu.async_copy(rb0, xg_hbm.at[pl.ds(gbase + 2 * GC, GC)], ws0)
    w1.wait()
    w2.wait()

    _scopeE.__exit__(None, None, None)

    @pl.when((c == 0) & (s == 0))
    def _():
        be0 = zero16
        be1 = zero16
        st0 = iota * BR
        st1 = (iota + L) * BR
        for e in range(E):
            e16 = jnp.full((L,), ends[e], jnp.int32)
            be0 = be0 + jnp.where(st0 >= e16, 1, 0)
            be1 = be1 + jnp.where(st1 >= e16, 1, 0)
        bexpv[pl.ds(0, L)] = jnp.minimum(be0, E - 1)
        bexpv[pl.ds(L, L)] = jnp.minimum(be1, E - 1)
        pltpu.sync_copy(bexpv, bexp_hbm)
        nblkv[...] = jnp.full((L,), nblocks, jnp.int32)
        pltpu.sync_copy(nblkv.at[pl.ds(0, 8)], nblk_hbm)


def _mlp_body(bexp_ref, nblk_ref, xg_ref, wgu_ref, bgu_g_ref,
              bgu_u_ref, wd_ref, bd_ref, wrow_ref, yg_ref):
    i = pl.program_id(0)

    @pl.when(i < nblk_ref[0])
    def _():
        xb = xg_ref[...].astype(jnp.float32)
        wgu = wgu_ref[0]
        wg_ = wgu[:, :D]
        wu_ = wgu[:, D:]
        gate = jax.lax.dot_general(
            xb, wg_, (((1,), (1,)), ((), ())),
            preferred_element_type=jnp.float32) + bgu_g_ref[0]
        up = jax.lax.dot_general(
            xb, wu_, (((1,), (1,)), ((), ())),
            preferred_element_type=jnp.float32) + bgu_u_ref[0]
        gate = jnp.minimum(gate, LIMIT)
        up = jnp.clip(up, -LIMIT, LIMIT)
        glu = gate * jax.nn.sigmoid(gate * ALPHA)
        h = (up + 1.0) * glu
        y = jax.lax.dot_general(
            h, wd_ref[0], (((1,), (1,)), ((), ())),
            preferred_element_type=jnp.float32) + bd_ref[0]
        yg_ref[...] = y * wrow_ref[0]


def _combine_body(yg_hbm, pos_hbm, out_hbm, posv, buf, obuf, sem):
    c = lax.axis_index("c")
    s = lax.axis_index("s")
    wid = s * NC + c
    pltpu.sync_copy(pos_hbm.at[pl.ds(wid * TPW * K, TPW * K)], posv)
    for ch in range(2):
        pltpu.async_copy(
            yg_hbm.at[posv.at[pl.ds(ch * TPW, TPW)]], buf, sem).wait()

        def body_r(r, carry):
            for u in range(D // L):
                a = buf[2 * r, pl.ds(u * L, L)]
                b = buf[2 * r + 1, pl.ds(u * L, L)]
                obuf[r, pl.ds(u * L, L)] = a + b
            return carry

        lax.fori_loop(0, TPW // 2, body_r, 0)
        pltpu.sync_copy(
            obuf, out_hbm.at[pl.ds(wid * TPW + ch * (TPW // 2), TPW // 2)])


def kernel(hidden_states, Wg, bg, Wgu, bgu, Wd, bd):
    x = hidden_states.reshape(S, D)

    sel, w = pl.pallas_call(
        _router_body,
        out_shape=(
            jax.ShapeDtypeStruct((S, K), jnp.int32),
            jax.ShapeDtypeStruct((S, K), jnp.float32),
        ),
    )(x, Wg, bg.reshape(1, E))

    mesh = plsc.VectorSubcoreMesh(core_axis_name="c", subcore_axis_name="s")
    xg, wmap, pos, bexp, nblk = pl.kernel(
        _dispatch_body,
        out_type=(
            jax.ShapeDtypeStruct((NR, D // 2), jnp.int32),
            jax.ShapeDtypeStruct((NR,), jnp.float32),
            jax.ShapeDtypeStruct((P,), jnp.int32),
            jax.ShapeDtypeStruct((2 * L,), jnp.int32),
            jax.ShapeDtypeStruct((8,), jnp.int32),
        ),
        mesh=mesh,
        scratch_types=[
            pltpu.VMEM((PW,), jnp.int32),
            pltpu.VMEM((PW,), jnp.float32),
            pltpu.VMEM((PW,), jnp.int32),
            pltpu.VMEM((PW,), jnp.int32),
            pltpu.VMEM((PW // 2,), jnp.int32),
            pltpu.VMEM((PW // 2,), jnp.int32),
            pltpu.VMEM((L,), jnp.int32),
            pltpu.VMEM((NSUB * L,), jnp.int32),
            pltpu.VMEM((NR // NSUB,), jnp.int32),
            pltpu.VMEM((NR // NSUB,), jnp.float32),
            pltpu.VMEM((GR,), jnp.int32),
            pltpu.VMEM((GR,), jnp.float32),
            pltpu.VMEM((GC, D // 2), jnp.int32),
            pltpu.VMEM((GC, D // 2), jnp.int32),
            pltpu.VMEM((2 * L,), jnp.int32),
            pltpu.VMEM((L,), jnp.int32),
            pltpu.VMEM_SHARED((NSUB * L,), jnp.int32),
            pltpu.VMEM_SHARED((NR,), jnp.int32),
            pltpu.VMEM_SHARED((NR,), jnp.float32),
            pltpu.SemaphoreType.DMA,
            pltpu.SemaphoreType.DMA,
            pltpu.SemaphoreType.DMA,
            pltpu.SemaphoreType.DMA,
        ],
    )(sel.reshape(P), w.reshape(P),
      lax.bitcast_convert_type(
          x.astype(jnp.bfloat16).reshape(S, D // 2, 2), jnp.int32))

    bgu_g = bgu[:, 0::2].reshape(E, 1, F)
    bgu_u = bgu[:, 1::2].reshape(E, 1, F)

    grid_spec = pltpu.PrefetchScalarGridSpec(
        num_scalar_prefetch=2,
        grid=(NB,),
        in_specs=[
            pl.BlockSpec((BR, D), lambda i, be, nb: (i, 0)),
            pl.BlockSpec((1, F, 2 * D), lambda i, be, nb: (be[i], 0, 0)),
            pl.BlockSpec((1, 1, F), lambda i, be, nb: (be[i], 0, 0)),
            pl.BlockSpec((1, 1, F), lambda i, be, nb: (be[i], 0, 0)),
            pl.BlockSpec((1, D, F), lambda i, be, nb: (be[i], 0, 0)),
            pl.BlockSpec((1, 1, D), lambda i, be, nb: (be[i], 0, 0)),
            pl.BlockSpec((1, BR, 1), lambda i, be, nb: (i, 0, 0)),
        ],
        out_specs=pl.BlockSpec((BR, D), lambda i, be, nb: (i, 0)),
    )
    xg_bf = lax.bitcast_convert_type(xg, jnp.bfloat16).reshape(NR, D)
    yg = pl.pallas_call(
        _mlp_body,
        grid_spec=grid_spec,
        out_shape=jax.ShapeDtypeStruct((NR, D), jnp.float32),
    )(bexp, nblk, xg_bf, Wgu.reshape(E, F, 2 * D),
      bgu_g, bgu_u, Wd, bd.reshape(E, 1, D), wmap.reshape(NB, BR, 1))

    out = pl.kernel(
        _combine_body,
        out_type=jax.ShapeDtypeStruct((S, D), jnp.float32),
        mesh=plsc.VectorSubcoreMesh(core_axis_name="c",
                                    subcore_axis_name="s"),
        scratch_types=[
            pltpu.VMEM((TPW * K,), jnp.int32),
            pltpu.VMEM((TPW, D), jnp.float32),
            pltpu.VMEM((TPW // 2, D), jnp.float32),
            pltpu.SemaphoreType.DMA,
        ],
    )(yg, pos)

    return out.reshape(B, S, D)


# speedup vs baseline: 1.6015x; 1.6015x over previous
import jax
import jax.numpy as jnp
from jax import lax
from jax.experimental import pallas as pl
from jax.experimental.pallas import tpu as pltpu
from jax.experimental.pallas import tpu_sc as plsc

B, S, D = 1, 2048, 768
E, K, F = 8, 2, 768
ALPHA, LIMIT = 1.702, 7.0

P = S * K
BR = 256
NB = P // BR + E
NR = NB * BR
L = 16
NC, NSUB = 2, 16
NW = NC * NSUB
PW = P // NSUB
VPW = PW // L
GR = NR // NW
GC = 64
TPW = S // NW
ZV = (NR // NSUB) // L


def _router_body(x_ref, wg_ref, bg_ref, sel_ref, w_ref, xp_ref):
    x = x_ref[...]
    logits = jax.lax.dot_general(
        x, wg_ref[...], (((1,), (1,)), ((), ())),
        preferred_element_type=jnp.float32)
    logits = logits + bg_ref[...]
    idx8 = jax.lax.broadcasted_iota(jnp.int32, (S, E), 1)
    m1 = jnp.max(logits, axis=1, keepdims=True)
    a1 = jnp.min(jnp.where(logits == m1, idx8, E), axis=1, keepdims=True)
    masked = jnp.where(idx8 == a1, -jnp.inf, logits)
    m2 = jnp.max(masked, axis=1, keepdims=True)
    a2 = jnp.min(jnp.where(masked == m2, idx8, E), axis=1, keepdims=True)
    w1 = jax.nn.sigmoid(m1 - m2)
    sel_ref[...] = jnp.concatenate([a1, a2], axis=1)
    w_ref[...] = jnp.concatenate([w1, 1.0 - w1], axis=1)
    xi = jax.lax.bitcast_convert_type(x, jnp.int32) + jnp.int32(0x8000)
    lo = (xi[:, :D // 2] >> 16) & jnp.int32(0xFFFF)
    hi = xi[:, D // 2:] & jnp.int32(-65536)
    xp_ref[...] = lo | hi


def _dispatch_body(sel_hbm, wp_hbm, x_hbm, xg_hbm, wmap_hbm, pos_hbm,
                   bexp_hbm, nblk_hbm, keyv, wv, rankv, tokv, posA, posB,
                   cnt_tbl, counts_all, zb_i, zb_f, idxv, wsl, rb0, rb1,
                   bexpv, nblkv, counts_sh, tok_sh, w_sh,
                   gs0, gs1, ws0, ws1):
    c = lax.axis_index("c")
    s = lax.axis_index("s")
    wid = s * NC + c
    pbase = s * PW
    iota = lax.iota(jnp.int32, L)
    zero16 = jnp.zeros((L,), jnp.int32)

    _scopeA = jax.named_scope("phaseA_sort"); _scopeA.__enter__()
    pltpu.sync_copy(sel_hbm.at[pl.ds(pbase, PW)], keyv)
    pltpu.sync_copy(wp_hbm.at[pl.ds(pbase, PW)], wv)
    run16 = zero16
    for j in range(VPW):
        k16 = keyv[pl.ds(j * L, L)]
        rank16 = zero16
        cnt16 = zero16
        for l in range(L):
            kl = k16[l]
            mask_l = jnp.where(iota > l, 1, 0)
            rank16 = rank16 + jnp.where(k16 == kl, 1, 0) * mask_l
            cnt16 = cnt16 + jnp.where(iota == kl, 1, 0)
        prev16 = zero16
        for e in range(E):
            prev16 = prev16 + jnp.where(
                k16 == e, jnp.full((L,), run16[e], jnp.int32), 0)
        rankv[pl.ds(j * L, L)] = prev16 + rank16
        run16 = run16 + cnt16
    cnt_tbl[...] = run16
    pltpu.sync_copy(cnt_tbl, counts_sh.at[pl.ds(s * L, L)])
    plsc.subcore_barrier()
    _scopeA.__exit__(None, None, None)
    _scopeB = jax.named_scope("phaseBC_pos"); _scopeB.__enter__()

    pltpu.sync_copy(counts_sh, counts_all)
    totals = zero16
    prefix = zero16
    for w2 in range(NSUB):
        c16 = counts_all[pl.ds(w2 * L, L)]
        m = -((w2 - s) >> 31)
        prefix = prefix + c16 * jnp.full((L,), m, jnp.int32)
        totals = totals + c16
    padded = ((totals + (BR - 1)) >> 8) << 8
    ends = zero16
    for e in range(E):
        ends = ends + jnp.where(
            iota >= e, jnp.full((L,), padded[e], jnp.int32), 0)
    offs = ends - padded
    base16 = offs + prefix
    base_sc = [base16[e] for e in range(E)]
    nblocks = ends[E - 1] >> 8

    for j in range(VPW):
        k16 = keyv[pl.ds(j * L, L)]
        b16 = zero16
        for e in range(E):
            b16 = b16 + jnp.where(
                k16 == e, jnp.full((L,), base_sc[e], jnp.int32), 0)
        p16 = b16 + rankv[pl.ds(j * L, L)]
        if j < VPW // 2:
            posA[pl.ds(j * L, L)] = p16
        else:
            posB[pl.ds((j - VPW // 2) * L, L)] = p16
        tokv[pl.ds(j * L, L)] = (
            jnp.full((L,), pbase + j * L, jnp.int32) + iota) >> 1

    @pl.when(c == 0)
    def _():
        pltpu.sync_copy(posA, pos_hbm.at[pl.ds(pbase, PW // 2)])
        pltpu.sync_copy(posB, pos_hbm.at[pl.ds(pbase + PW // 2, PW // 2)])

    _scopeB.__exit__(None, None, None)
    _scopeD = jax.named_scope("phaseD_scatter"); _scopeD.__enter__()
    zf16 = jnp.zeros((L,), jnp.float32)
    for j in range(ZV):
        zb_i[pl.ds(j * L, L)] = zero16
        zb_f[pl.ds(j * L, L)] = zf16
    pltpu.sync_copy(zb_i, tok_sh.at[pl.ds(s * (NR // NSUB), NR // NSUB)])
    pltpu.sync_copy(zb_f, w_sh.at[pl.ds(s * (NR // NSUB), NR // NSUB)])
    plsc.subcore_barrier()
    pltpu.sync_copy(tokv.at[pl.ds(0, PW // 2)], tok_sh.at[posA])
    pltpu.sync_copy(tokv.at[pl.ds(PW // 2, PW // 2)], tok_sh.at[posB])
    pltpu.sync_copy(wv.at[pl.ds(0, PW // 2)], w_sh.at[posA])
    pltpu.sync_copy(wv.at[pl.ds(PW // 2, PW // 2)], w_sh.at[posB])
    plsc.subcore_barrier()

    _scopeD.__exit__(None, None, None)
    _scopeE = jax.named_scope("phaseE_gather"); _scopeE.__enter__()
    gbase = wid * GR
    pltpu.sync_copy(tok_sh.at[pl.ds(gbase, GR)], idxv)
    pltpu.sync_copy(w_sh.at[pl.ds(gbase, GR)], wsl)
    pltpu.sync_copy(wsl, wmap_hbm.at[pl.ds(gbase, GR)])
    g0 = pltpu.async_copy(x_hbm.at[idxv.at[pl.ds(0, GC)]], rb0, gs0)
    g0.wait()
    g1 = pltpu.async_copy(x_hbm.at[idxv.at[pl.ds(GC, GC)]], rb1, gs1)
    w0 = pltpu.async_copy(rb0, xg_hbm.at[pl.ds(gbase, GC)], ws0)
    g1.wait()
    w0.wait()
    g2 = pltpu.async_copy(x_hbm.at[idxv.at[pl.ds(2 * GC, GC)]], rb0, gs0)
    w1 = pltpu.async_copy(rb1, xg_hbm.at[pl.ds(gbase + GC, GC)], ws1)
    g2.wait()
    w2 = pltpu.async_copy(rb0, xg_hbm.at[pl.ds(gbase + 2 * GC, GC)], ws0)
    w1.wait()
    w2.wait()

    _scopeE.__exit__(None, None, None)

    @pl.when((c == 0) & (s == 0))
    def _():
        be0 = zero16
        be1 = zero16
        st0 = iota * BR
        st1 = (iota + L) * BR
        for e in range(E):
            e16 = jnp.full((L,), ends[e], jnp.int32)
            be0 = be0 + jnp.where(st0 >= e16, 1, 0)
            be1 = be1 + jnp.where(st1 >= e16, 1, 0)
        bexpv[pl.ds(0, L)] = jnp.minimum(be0, E - 1)
        bexpv[pl.ds(L, L)] = jnp.minimum(be1, E - 1)
        pltpu.sync_copy(bexpv, bexp_hbm)
        nblkv[...] = jnp.full((L,), nblocks, jnp.int32)
        pltpu.sync_copy(nblkv.at[pl.ds(0, 8)], nblk_hbm)


def _mlp_body(bexp_ref, nblk_ref, xg_ref, wgu_ref, bgu_g_ref,
              bgu_u_ref, wd_ref, bd_ref, wrow_ref, yg_ref):
    i = pl.program_id(0)

    @pl.when(i < nblk_ref[0])
    def _():
        xgi = xg_ref[...]
        lo_f = jax.lax.bitcast_convert_type(xgi << 16, jnp.float32)
        hi_f = jax.lax.bitcast_convert_type(
            xgi & jnp.int32(-65536), jnp.float32)
        xb = jnp.concatenate([lo_f, hi_f], axis=1)
        wgu = wgu_ref[0]
        wg_ = wgu[:, :D]
        wu_ = wgu[:, D:]
        gate = jax.lax.dot_general(
            xb, wg_, (((1,), (1,)), ((), ())),
            preferred_element_type=jnp.float32) + bgu_g_ref[0]
        up = jax.lax.dot_general(
            xb, wu_, (((1,), (1,)), ((), ())),
            preferred_element_type=jnp.float32) + bgu_u_ref[0]
        gate = jnp.minimum(gate, LIMIT)
        up = jnp.clip(up, -LIMIT, LIMIT)
        glu = gate * jax.nn.sigmoid(gate * ALPHA)
        h = (up + 1.0) * glu
        y = jax.lax.dot_general(
            h, wd_ref[0], (((1,), (1,)), ((), ())),
            preferred_element_type=jnp.float32) + bd_ref[0]
        yg_ref[...] = y * wrow_ref[0]


def _combine_body(yg_hbm, pos_hbm, out_hbm, posv, buf, obuf, sem):
    c = lax.axis_index("c")
    s = lax.axis_index("s")
    wid = s * NC + c
    pltpu.sync_copy(pos_hbm.at[pl.ds(wid * TPW * K, TPW * K)], posv)
    for ch in range(2):
        pltpu.async_copy(
            yg_hbm.at[posv.at[pl.ds(ch * TPW, TPW)]], buf, sem).wait()

        def body_r(r, carry):
            for u in range(D // L):
                a = buf[2 * r, pl.ds(u * L, L)]
                b = buf[2 * r + 1, pl.ds(u * L, L)]
                obuf[r, pl.ds(u * L, L)] = a + b
            return carry

        lax.fori_loop(0, TPW // 2, body_r, 0)
        pltpu.sync_copy(
            obuf, out_hbm.at[pl.ds(wid * TPW + ch * (TPW // 2), TPW // 2)])


def kernel(hidden_states, Wg, bg, Wgu, bgu, Wd, bd):
    x = hidden_states.reshape(S, D)

    sel, w, xp = pl.pallas_call(
        _router_body,
        out_shape=(
            jax.ShapeDtypeStruct((S, K), jnp.int32),
            jax.ShapeDtypeStruct((S, K), jnp.float32),
            jax.ShapeDtypeStruct((S, D // 2), jnp.int32),
        ),
    )(x, Wg, bg.reshape(1, E))

    mesh = plsc.VectorSubcoreMesh(core_axis_name="c", subcore_axis_name="s")
    xg, wmap, pos, bexp, nblk = pl.kernel(
        _dispatch_body,
        out_type=(
            jax.ShapeDtypeStruct((NR, D // 2), jnp.int32),
            jax.ShapeDtypeStruct((NR,), jnp.float32),
            jax.ShapeDtypeStruct((P,), jnp.int32),
            jax.ShapeDtypeStruct((2 * L,), jnp.int32),
            jax.ShapeDtypeStruct((8,), jnp.int32),
        ),
        mesh=mesh,
        scratch_types=[
            pltpu.VMEM((PW,), jnp.int32),
            pltpu.VMEM((PW,), jnp.float32),
            pltpu.VMEM((PW,), jnp.int32),
            pltpu.VMEM((PW,), jnp.int32),
            pltpu.VMEM((PW // 2,), jnp.int32),
            pltpu.VMEM((PW // 2,), jnp.int32),
            pltpu.VMEM((L,), jnp.int32),
            pltpu.VMEM((NSUB * L,), jnp.int32),
            pltpu.VMEM((NR // NSUB,), jnp.int32),
            pltpu.VMEM((NR // NSUB,), jnp.float32),
            pltpu.VMEM((GR,), jnp.int32),
            pltpu.VMEM((GR,), jnp.float32),
            pltpu.VMEM((GC, D // 2), jnp.int32),
            pltpu.VMEM((GC, D // 2), jnp.int32),
            pltpu.VMEM((2 * L,), jnp.int32),
            pltpu.VMEM((L,), jnp.int32),
            pltpu.VMEM_SHARED((NSUB * L,), jnp.int32),
            pltpu.VMEM_SHARED((NR,), jnp.int32),
            pltpu.VMEM_SHARED((NR,), jnp.float32),
            pltpu.SemaphoreType.DMA,
            pltpu.SemaphoreType.DMA,
            pltpu.SemaphoreType.DMA,
            pltpu.SemaphoreType.DMA,
        ],
    )(sel.reshape(P), w.reshape(P), xp)

    bgu_g = bgu[:, 0::2].reshape(E, 1, F)
    bgu_u = bgu[:, 1::2].reshape(E, 1, F)

    grid_spec = pltpu.PrefetchScalarGridSpec(
        num_scalar_prefetch=2,
        grid=(NB,),
        in_specs=[
            pl.BlockSpec((BR, D // 2), lambda i, be, nb: (i, 0)),
            pl.BlockSpec((1, F, 2 * D), lambda i, be, nb: (be[i], 0, 0)),
            pl.BlockSpec((1, 1, F), lambda i, be, nb: (be[i], 0, 0)),
            pl.BlockSpec((1, 1, F), lambda i, be, nb: (be[i], 0, 0)),
            pl.BlockSpec((1, D, F), lambda i, be, nb: (be[i], 0, 0)),
            pl.BlockSpec((1, 1, D), lambda i, be, nb: (be[i], 0, 0)),
            pl.BlockSpec((1, BR, 1), lambda i, be, nb: (i, 0, 0)),
        ],
        out_specs=pl.BlockSpec((BR, D), lambda i, be, nb: (i, 0)),
    )
    yg = pl.pallas_call(
        _mlp_body,
        grid_spec=grid_spec,
        out_shape=jax.ShapeDtypeStruct((NR, D), jnp.float32),
    )(bexp, nblk, xg, Wgu.reshape(E, F, 2 * D),
      bgu_g, bgu_u, Wd, bd.reshape(E, 1, D), wmap.reshape(NB, BR, 1))

    out = pl.kernel(
        _combine_body,
        out_type=jax.ShapeDtypeStruct((S, D), jnp.float32),
        mesh=plsc.VectorSubcoreMesh(core_axis_name="c",
                                    subcore_axis_name="s"),
        scratch_types=[
            pltpu.VMEM((TPW * K,), jnp.int32),
            pltpu.VMEM((TPW, D), jnp.float32),
            pltpu.VMEM((TPW // 2, D), jnp.float32),
            pltpu.SemaphoreType.DMA,
        ],
    )(yg, pos)

    return out.reshape(B, S, D)


# speedup vs baseline: 1.7705x; 1.1055x over previous
import jax
import jax.numpy as jnp
from jax import lax
from jax.experimental import pallas as pl
from jax.experimental.pallas import tpu as pltpu
from jax.experimental.pallas import tpu_sc as plsc

B, S, D = 1, 2048, 768
E, K, F = 8, 2, 768
ALPHA, LIMIT = 1.702, 7.0

P = S * K
BR = 128
SH = 7
NB = P // BR + E
NR = NB * BR
L = 16
NC, NSUB = 2, 16
NW = NC * NSUB
PW = P // NSUB
VPW = PW // L
GR = NR // NW
GC = 80
TPW = S // NW
ZV = (NR // NSUB) // L


def _router_body(x_ref, wg_ref, bg_ref, sel_ref, w_ref, xp_ref):
    x = x_ref[...]
    logits = jax.lax.dot_general(
        x, wg_ref[...], (((1,), (1,)), ((), ())),
        preferred_element_type=jnp.float32)
    logits = logits + bg_ref[...]
    idx8 = jax.lax.broadcasted_iota(jnp.int32, (S, E), 1)
    m1 = jnp.max(logits, axis=1, keepdims=True)
    a1 = jnp.min(jnp.where(logits == m1, idx8, E), axis=1, keepdims=True)
    masked = jnp.where(idx8 == a1, -jnp.inf, logits)
    m2 = jnp.max(masked, axis=1, keepdims=True)
    a2 = jnp.min(jnp.where(masked == m2, idx8, E), axis=1, keepdims=True)
    w1 = jax.nn.sigmoid(m1 - m2)
    sel_ref[...] = jnp.concatenate([a1, a2], axis=1)
    w_ref[...] = jnp.concatenate([w1, 1.0 - w1], axis=1)
    xi = jax.lax.bitcast_convert_type(x, jnp.int32) + jnp.int32(0x8000)
    lo = (xi[:, :D // 2] >> 16) & jnp.int32(0xFFFF)
    hi = xi[:, D // 2:] & jnp.int32(-65536)
    xp_ref[...] = lo | hi


def _dispatch_body(sel_hbm, wp_hbm, x_hbm, xg_hbm, wmap_hbm, pos_hbm,
                   bexp_hbm, nblk_hbm, keyv, wv, rankv, tokv, posA, posB,
                   cnt_tbl, counts_all, zb_i, zb_f, idxv, wsl, rb0, rb1,
                   bexpv, nblkv, counts_sh, tok_sh, w_sh,
                   gs0, gs1, ws0, ws1):
    c = lax.axis_index("c")
    s = lax.axis_index("s")
    wid = s * NC + c
    pbase = s * PW
    iota = lax.iota(jnp.int32, L)
    zero16 = jnp.zeros((L,), jnp.int32)

    _scopeA = jax.named_scope("phaseA_sort"); _scopeA.__enter__()
    pltpu.sync_copy(sel_hbm.at[pl.ds(pbase, PW)], keyv)
    pltpu.sync_copy(wp_hbm.at[pl.ds(pbase, PW)], wv)
    run16 = zero16
    for j in range(VPW):
        k16 = keyv[pl.ds(j * L, L)]
        rank16 = zero16
        cnt16 = zero16
        for l in range(L):
            kl = k16[l]
            mask_l = jnp.where(iota > l, 1, 0)
            rank16 = rank16 + jnp.where(k16 == kl, 1, 0) * mask_l
            cnt16 = cnt16 + jnp.where(iota == kl, 1, 0)
        prev16 = zero16
        for e in range(E):
            prev16 = prev16 + jnp.where(
                k16 == e, jnp.full((L,), run16[e], jnp.int32), 0)
        rankv[pl.ds(j * L, L)] = prev16 + rank16
        run16 = run16 + cnt16
    cnt_tbl[...] = run16
    pltpu.sync_copy(cnt_tbl, counts_sh.at[pl.ds(s * L, L)])
    plsc.subcore_barrier()
    _scopeA.__exit__(None, None, None)
    _scopeB = jax.named_scope("phaseBC_pos"); _scopeB.__enter__()

    pltpu.sync_copy(counts_sh, counts_all)
    totals = zero16
    prefix = zero16
    for w2 in range(NSUB):
        c16 = counts_all[pl.ds(w2 * L, L)]
        m = -((w2 - s) >> 31)
        prefix = prefix + c16 * jnp.full((L,), m, jnp.int32)
        totals = totals + c16
    padded = ((totals + (BR - 1)) >> SH) << SH
    ends = zero16
    for e in range(E):
        ends = ends + jnp.where(
            iota >= e, jnp.full((L,), padded[e], jnp.int32), 0)
    offs = ends - padded
    base16 = offs + prefix
    base_sc = [base16[e] for e in range(E)]
    nblocks = ends[E - 1] >> SH

    for j in range(VPW):
        k16 = keyv[pl.ds(j * L, L)]
        b16 = zero16
        for e in range(E):
            b16 = b16 + jnp.where(
                k16 == e, jnp.full((L,), base_sc[e], jnp.int32), 0)
        p16 = b16 + rankv[pl.ds(j * L, L)]
        if j < VPW // 2:
            posA[pl.ds(j * L, L)] = p16
        else:
            posB[pl.ds((j - VPW // 2) * L, L)] = p16
        tokv[pl.ds(j * L, L)] = (
            jnp.full((L,), pbase + j * L, jnp.int32) + iota) >> 1

    @pl.when(c == 0)
    def _():
        pltpu.sync_copy(posA, pos_hbm.at[pl.ds(pbase, PW // 2)])
        pltpu.sync_copy(posB, pos_hbm.at[pl.ds(pbase + PW // 2, PW // 2)])

    _scopeB.__exit__(None, None, None)
    _scopeD = jax.named_scope("phaseD_scatter"); _scopeD.__enter__()
    zf16 = jnp.zeros((L,), jnp.float32)
    for j in range(ZV):
        zb_i[pl.ds(j * L, L)] = zero16
        zb_f[pl.ds(j * L, L)] = zf16
    pltpu.sync_copy(zb_i, tok_sh.at[pl.ds(s * (NR // NSUB), NR // NSUB)])
    pltpu.sync_copy(zb_f, w_sh.at[pl.ds(s * (NR // NSUB), NR // NSUB)])
    plsc.subcore_barrier()
    pltpu.sync_copy(tokv.at[pl.ds(0, PW // 2)], tok_sh.at[posA])
    pltpu.sync_copy(tokv.at[pl.ds(PW // 2, PW // 2)], tok_sh.at[posB])
    pltpu.sync_copy(wv.at[pl.ds(0, PW // 2)], w_sh.at[posA])
    pltpu.sync_copy(wv.at[pl.ds(PW // 2, PW // 2)], w_sh.at[posB])
    plsc.subcore_barrier()

    _scopeD.__exit__(None, None, None)
    _scopeE = jax.named_scope("phaseE_gather"); _scopeE.__enter__()
    gbase = wid * GR
    pltpu.sync_copy(tok_sh.at[pl.ds(gbase, GR)], idxv)
    pltpu.sync_copy(w_sh.at[pl.ds(gbase, GR)], wsl)
    pltpu.sync_copy(wsl, wmap_hbm.at[pl.ds(gbase, GR)])
    g0 = pltpu.async_copy(x_hbm.at[idxv.at[pl.ds(0, GC)]], rb0, gs0)
    g0.wait()
    g1 = pltpu.async_copy(x_hbm.at[idxv.at[pl.ds(GC, GC)]], rb1, gs1)
    w0 = pltpu.async_copy(rb0, xg_hbm.at[pl.ds(gbase, GC)], ws0)
    g1.wait()
    w1 = pltpu.async_copy(rb1, xg_hbm.at[pl.ds(gbase + GC, GC)], ws1)
    w0.wait()
    w1.wait()

    _scopeE.__exit__(None, None, None)

    @pl.when((c == 0) & (s == 0))
    def _():
        for v in range(3):
            bev = zero16
            stv = (iota + v * L) * BR
            for e in range(E):
                e16 = jnp.full((L,), ends[e], jnp.int32)
                bev = bev + jnp.where(stv >= e16, 1, 0)
            bexpv[pl.ds(v * L, L)] = jnp.minimum(bev, E - 1)
        pltpu.sync_copy(bexpv, bexp_hbm)
        nblkv[...] = jnp.full((L,), nblocks, jnp.int32)
        pltpu.sync_copy(nblkv.at[pl.ds(0, 8)], nblk_hbm)


def _mlp_body(bexp_ref, nblk_ref, xg_ref, wgu_ref, bgu_g_ref,
              bgu_u_ref, wd_ref, bd_ref, wrow_ref, yg_ref):
    i = pl.program_id(0)

    @pl.when(i < nblk_ref[0])
    def _():
        xgi = xg_ref[...]
        lo_f = jax.lax.bitcast_convert_type(xgi << 16, jnp.float32)
        hi_f = jax.lax.bitcast_convert_type(
            xgi & jnp.int32(-65536), jnp.float32)
        xb = jnp.concatenate([lo_f, hi_f], axis=1)
        wgu = wgu_ref[0]
        wg_ = wgu[:, :D]
        wu_ = wgu[:, D:]
        gate = jax.lax.dot_general(
            xb, wg_, (((1,), (1,)), ((), ())),
            preferred_element_type=jnp.float32) + bgu_g_ref[0]
        up = jax.lax.dot_general(
            xb, wu_, (((1,), (1,)), ((), ())),
            preferred_element_type=jnp.float32) + bgu_u_ref[0]
        gate = jnp.minimum(gate, LIMIT)
        up = jnp.clip(up, -LIMIT, LIMIT)
        glu = gate * jax.nn.sigmoid(gate * ALPHA)
        h = (up + 1.0) * glu
        y = jax.lax.dot_general(
            h, wd_ref[0], (((1,), (1,)), ((), ())),
            preferred_element_type=jnp.float32) + bd_ref[0]
        yg_ref[...] = y * wrow_ref[0]


def _combine_body(yg_hbm, pos_hbm, out_hbm, posv, buf, obuf, sem):
    c = lax.axis_index("c")
    s = lax.axis_index("s")
    wid = s * NC + c
    pltpu.sync_copy(pos_hbm.at[pl.ds(wid * TPW * K, TPW * K)], posv)
    for ch in range(2):
        pltpu.async_copy(
            yg_hbm.at[posv.at[pl.ds(ch * TPW, TPW)]], buf, sem).wait()

        def body_r(r, carry):
            for u in range(D // L):
                a = buf[2 * r, pl.ds(u * L, L)]
                b = buf[2 * r + 1, pl.ds(u * L, L)]
                obuf[r, pl.ds(u * L, L)] = a + b
            return carry

        lax.fori_loop(0, TPW // 2, body_r, 0)
        pltpu.sync_copy(
            obuf, out_hbm.at[pl.ds(wid * TPW + ch * (TPW // 2), TPW // 2)])


def kernel(hidden_states, Wg, bg, Wgu, bgu, Wd, bd):
    x = hidden_states.reshape(S, D)

    sel, w, xp = pl.pallas_call(
        _router_body,
        out_shape=(
            jax.ShapeDtypeStruct((S, K), jnp.int32),
            jax.ShapeDtypeStruct((S, K), jnp.float32),
            jax.ShapeDtypeStruct((S, D // 2), jnp.int32),
        ),
    )(x, Wg, bg.reshape(1, E))

    mesh = plsc.VectorSubcoreMesh(core_axis_name="c", subcore_axis_name="s")
    xg, wmap, pos, bexp, nblk = pl.kernel(
        _dispatch_body,
        out_type=(
            jax.ShapeDtypeStruct((NR, D // 2), jnp.int32),
            jax.ShapeDtypeStruct((NR,), jnp.float32),
            jax.ShapeDtypeStruct((P,), jnp.int32),
            jax.ShapeDtypeStruct((3 * L,), jnp.int32),
            jax.ShapeDtypeStruct((8,), jnp.int32),
        ),
        mesh=mesh,
        scratch_types=[
            pltpu.VMEM((PW,), jnp.int32),
            pltpu.VMEM((PW,), jnp.float32),
            pltpu.VMEM((PW,), jnp.int32),
            pltpu.VMEM((PW,), jnp.int32),
            pltpu.VMEM((PW // 2,), jnp.int32),
            pltpu.VMEM((PW // 2,), jnp.int32),
            pltpu.VMEM((L,), jnp.int32),
            pltpu.VMEM((NSUB * L,), jnp.int32),
            pltpu.VMEM((NR // NSUB,), jnp.int32),
            pltpu.VMEM((NR // NSUB,), jnp.float32),
            pltpu.VMEM((GR,), jnp.int32),
            pltpu.VMEM((GR,), jnp.float32),
            pltpu.VMEM((GC, D // 2), jnp.int32),
            pltpu.VMEM((GC, D // 2), jnp.int32),
            pltpu.VMEM((3 * L,), jnp.int32),
            pltpu.VMEM((L,), jnp.int32),
            pltpu.VMEM_SHARED((NSUB * L,), jnp.int32),
            pltpu.VMEM_SHARED((NR,), jnp.int32),
            pltpu.VMEM_SHARED((NR,), jnp.float32),
            pltpu.SemaphoreType.DMA,
            pltpu.SemaphoreType.DMA,
            pltpu.SemaphoreType.DMA,
            pltpu.SemaphoreType.DMA,
        ],
    )(sel.reshape(P), w.reshape(P), xp)

    bgu_g = bgu[:, 0::2].reshape(E, 1, F)
    bgu_u = bgu[:, 1::2].reshape(E, 1, F)

    grid_spec = pltpu.PrefetchScalarGridSpec(
        num_scalar_prefetch=2,
        grid=(NB,),
        in_specs=[
            pl.BlockSpec((BR, D // 2), lambda i, be, nb: (i, 0)),
            pl.BlockSpec((1, F, 2 * D), lambda i, be, nb: (be[i], 0, 0)),
            pl.BlockSpec((1, 1, F), lambda i, be, nb: (be[i], 0, 0)),
            pl.BlockSpec((1, 1, F), lambda i, be, nb: (be[i], 0, 0)),
            pl.BlockSpec((1, D, F), lambda i, be, nb: (be[i], 0, 0)),
            pl.BlockSpec((1, 1, D), lambda i, be, nb: (be[i], 0, 0)),
            pl.BlockSpec((1, BR, 1), lambda i, be, nb: (i, 0, 0)),
        ],
        out_specs=pl.BlockSpec((BR, D), lambda i, be, nb: (i, 0)),
    )
    yg = pl.pallas_call(
        _mlp_body,
        grid_spec=grid_spec,
        out_shape=jax.ShapeDtypeStruct((NR, D), jnp.float32),
    )(bexp, nblk, xg, Wgu.reshape(E, F, 2 * D),
      bgu_g, bgu_u, Wd, bd.reshape(E, 1, D), wmap.reshape(NB, BR, 1))

    out = pl.kernel(
        _combine_body,
        out_type=jax.ShapeDtypeStruct((S, D), jnp.float32),
        mesh=plsc.VectorSubcoreMesh(core_axis_name="c",
                                    subcore_axis_name="s"),
        scratch_types=[
            pltpu.VMEM((TPW * K,), jnp.int32),
            pltpu.VMEM((TPW, D), jnp.float32),
            pltpu.VMEM((TPW // 2, D), jnp.float32),
            pltpu.SemaphoreType.DMA,
        ],
    )(yg, pos)

    return out.reshape(B, S, D)


# speedup vs baseline: 1.7757x; 1.0030x over previous
import jax
import jax.numpy as jnp
from jax import lax
from jax.experimental import pallas as pl
from jax.experimental.pallas import tpu as pltpu
from jax.experimental.pallas import tpu_sc as plsc

B, S, D = 1, 2048, 768
E, K, F = 8, 2, 768
ALPHA, LIMIT = 1.702, 7.0

P = S * K
BR = 128
SH = 7
NB = P // BR + E
NR = NB * BR
L = 16
NC, NSUB = 2, 16
NW = NC * NSUB
PW = P // NSUB
VPW = PW // L
GR = NR // NW
GC = 80
TPW = S // NW
ZV = (NR // NSUB) // L


def _router_body(x_ref, wg_ref, bg_ref, sel_ref, w_ref, xp_ref):
    x = x_ref[...]
    logits = jax.lax.dot_general(
        x, wg_ref[...], (((1,), (1,)), ((), ())),
        preferred_element_type=jnp.float32)
    logits = logits + bg_ref[...]
    idx8 = jax.lax.broadcasted_iota(jnp.int32, (S, E), 1)
    m1 = jnp.max(logits, axis=1, keepdims=True)
    a1 = jnp.min(jnp.where(logits == m1, idx8, E), axis=1, keepdims=True)
    masked = jnp.where(idx8 == a1, -jnp.inf, logits)
    m2 = jnp.max(masked, axis=1, keepdims=True)
    a2 = jnp.min(jnp.where(masked == m2, idx8, E), axis=1, keepdims=True)
    w1 = jax.nn.sigmoid(m1 - m2)
    sel_ref[...] = jnp.concatenate([a1, a2], axis=1)
    w_ref[...] = jnp.concatenate([w1, 1.0 - w1], axis=1)
    xi = jax.lax.bitcast_convert_type(x, jnp.int32) + jnp.int32(0x8000)
    lo = (xi[:, :D // 2] >> 16) & jnp.int32(0xFFFF)
    hi = xi[:, D // 2:] & jnp.int32(-65536)
    xp_ref[...] = lo | hi


def _dispatch_body(sel_hbm, wp_hbm, x_hbm, xg_hbm, wmap_hbm, pos_hbm,
                   bexp_hbm, nblk_hbm, keyv, wv, rankv, tokv, posA, posB,
                   cnt_tbl, counts_all, zb_i, zb_f, idxv, wsl, rb0, rb1,
                   bexpv, nblkv, counts_sh, tok_sh, w_sh,
                   gs0, gs1, ws0, ws1):
    c = lax.axis_index("c")
    s = lax.axis_index("s")
    wid = s * NC + c
    pbase = s * PW
    iota = lax.iota(jnp.int32, L)
    zero16 = jnp.zeros((L,), jnp.int32)

    pltpu.sync_copy(sel_hbm.at[pl.ds(pbase, PW)], keyv)
    pltpu.sync_copy(wp_hbm.at[pl.ds(pbase, PW)], wv)
    run16 = zero16
    for j in range(VPW):
        k16 = keyv[pl.ds(j * L, L)]
        rank16 = zero16
        cnt16 = zero16
        for l in range(L):
            kl = k16[l]
            mask_l = jnp.where(iota > l, 1, 0)
            rank16 = rank16 + jnp.where(k16 == kl, 1, 0) * mask_l
            cnt16 = cnt16 + jnp.where(iota == kl, 1, 0)
        prev16 = zero16
        for e in range(E):
            prev16 = prev16 + jnp.where(
                k16 == e, jnp.full((L,), run16[e], jnp.int32), 0)
        rankv[pl.ds(j * L, L)] = prev16 + rank16
        run16 = run16 + cnt16
    cnt_tbl[...] = run16
    pltpu.sync_copy(cnt_tbl, counts_sh.at[pl.ds(s * L, L)])
    plsc.subcore_barrier()

    pltpu.sync_copy(counts_sh, counts_all)
    totals = zero16
    prefix = zero16
    for w2 in range(NSUB):
        c16 = counts_all[pl.ds(w2 * L, L)]
        m = -((w2 - s) >> 31)
        prefix = prefix + c16 * jnp.full((L,), m, jnp.int32)
        totals = totals + c16
    padded = ((totals + (BR - 1)) >> SH) << SH
    ends = zero16
    for e in range(E):
        ends = ends + jnp.where(
            iota >= e, jnp.full((L,), padded[e], jnp.int32), 0)
    offs = ends - padded
    base16 = offs + prefix
    base_sc = [base16[e] for e in range(E)]
    nblocks = ends[E - 1] >> SH

    for j in range(VPW):
        k16 = keyv[pl.ds(j * L, L)]
        b16 = zero16
        for e in range(E):
            b16 = b16 + jnp.where(
                k16 == e, jnp.full((L,), base_sc[e], jnp.int32), 0)
        p16 = b16 + rankv[pl.ds(j * L, L)]
        if j < VPW // 2:
            posA[pl.ds(j * L, L)] = p16
        else:
            posB[pl.ds((j - VPW // 2) * L, L)] = p16
        tokv[pl.ds(j * L, L)] = (
            jnp.full((L,), pbase + j * L, jnp.int32) + iota) >> 1

    @pl.when(c == 0)
    def _():
        pltpu.sync_copy(posA, pos_hbm.at[pl.ds(pbase, PW // 2)])
        pltpu.sync_copy(posB, pos_hbm.at[pl.ds(pbase + PW // 2, PW // 2)])

    zf16 = jnp.zeros((L,), jnp.float32)
    for j in range(ZV):
        zb_i[pl.ds(j * L, L)] = zero16
        zb_f[pl.ds(j * L, L)] = zf16
    pltpu.sync_copy(zb_i, tok_sh.at[pl.ds(s * (NR // NSUB), NR // NSUB)])
    pltpu.sync_copy(zb_f, w_sh.at[pl.ds(s * (NR // NSUB), NR // NSUB)])
    plsc.subcore_barrier()
    pltpu.sync_copy(tokv.at[pl.ds(0, PW // 2)], tok_sh.at[posA])
    pltpu.sync_copy(tokv.at[pl.ds(PW // 2, PW // 2)], tok_sh.at[posB])
    pltpu.sync_copy(wv.at[pl.ds(0, PW // 2)], w_sh.at[posA])
    pltpu.sync_copy(wv.at[pl.ds(PW // 2, PW // 2)], w_sh.at[posB])
    plsc.subcore_barrier()

    gbase = wid * GR
    pltpu.sync_copy(tok_sh.at[pl.ds(gbase, GR)], idxv)
    pltpu.sync_copy(w_sh.at[pl.ds(gbase, GR)], wsl)
    pltpu.sync_copy(wsl, wmap_hbm.at[pl.ds(gbase, GR)])
    g0 = pltpu.async_copy(x_hbm.at[idxv.at[pl.ds(0, GC)]], rb0, gs0)
    g0.wait()
    g1 = pltpu.async_copy(x_hbm.at[idxv.at[pl.ds(GC, GC)]], rb1, gs1)
    w0 = pltpu.async_copy(rb0, xg_hbm.at[pl.ds(gbase, GC)], ws0)
    g1.wait()
    w1 = pltpu.async_copy(rb1, xg_hbm.at[pl.ds(gbase + GC, GC)], ws1)
    w0.wait()
    w1.wait()


    @pl.when((c == 0) & (s == 0))
    def _():
        for v in range(3):
            bev = zero16
            stv = (iota + v * L) * BR
            for e in range(E):
                e16 = jnp.full((L,), ends[e], jnp.int32)
                bev = bev + jnp.where(stv >= e16, 1, 0)
            bexpv[pl.ds(v * L, L)] = jnp.minimum(bev, E - 1)
        pltpu.sync_copy(bexpv, bexp_hbm)
        nblkv[...] = jnp.full((L,), nblocks, jnp.int32)
        pltpu.sync_copy(nblkv.at[pl.ds(0, 8)], nblk_hbm)


def _mlp_body(bexp_ref, nblk_ref, xg_ref, wgu_ref, bgu_g_ref,
              bgu_u_ref, wd_ref, bd_ref, wrow_ref, yg_ref):
    i = pl.program_id(0)

    @pl.when(i < nblk_ref[0])
    def _():
        xgi = xg_ref[...]
        lo_f = jax.lax.bitcast_convert_type(xgi << 16, jnp.float32)
        hi_f = jax.lax.bitcast_convert_type(
            xgi & jnp.int32(-65536), jnp.float32)
        xb = jnp.concatenate([lo_f, hi_f], axis=1)
        wgu = wgu_ref[0]
        wg_ = wgu[:, :D]
        wu_ = wgu[:, D:]
        gate = jax.lax.dot_general(
            xb, wg_, (((1,), (1,)), ((), ())),
            preferred_element_type=jnp.float32) + bgu_g_ref[0]
        up = jax.lax.dot_general(
            xb, wu_, (((1,), (1,)), ((), ())),
            preferred_element_type=jnp.float32) + bgu_u_ref[0]
        gate = jnp.minimum(gate, LIMIT)
        up = jnp.clip(up, -LIMIT, LIMIT)
        glu = gate * jax.nn.sigmoid(gate * ALPHA)
        h = (up + 1.0) * glu
        y = jax.lax.dot_general(
            h, wd_ref[0], (((1,), (1,)), ((), ())),
            preferred_element_type=jnp.float32) + bd_ref[0]
        yg_ref[...] = y * wrow_ref[0]


def _combine_body(yg_hbm, pos_hbm, out_hbm, posv, buf, obuf, sem):
    c = lax.axis_index("c")
    s = lax.axis_index("s")
    wid = s * NC + c
    pltpu.sync_copy(pos_hbm.at[pl.ds(wid * TPW * K, TPW * K)], posv)
    for ch in range(2):
        pltpu.async_copy(
            yg_hbm.at[posv.at[pl.ds(ch * TPW, TPW)]], buf, sem).wait()

        def body_r(r, carry):
            for u in range(D // L):
                a = buf[2 * r, pl.ds(u * L, L)]
                b = buf[2 * r + 1, pl.ds(u * L, L)]
                obuf[r, pl.ds(u * L, L)] = a + b
            return carry

        lax.fori_loop(0, TPW // 2, body_r, 0)
        pltpu.sync_copy(
            obuf, out_hbm.at[pl.ds(wid * TPW + ch * (TPW // 2), TPW // 2)])


def kernel(hidden_states, Wg, bg, Wgu, bgu, Wd, bd):
    x = hidden_states.reshape(S, D)

    sel, w, xp = pl.pallas_call(
        _router_body,
        out_shape=(
            jax.ShapeDtypeStruct((S, K), jnp.int32),
            jax.ShapeDtypeStruct((S, K), jnp.float32),
            jax.ShapeDtypeStruct((S, D // 2), jnp.int32),
        ),
    )(x, Wg, bg.reshape(1, E))

    mesh = plsc.VectorSubcoreMesh(core_axis_name="c", subcore_axis_name="s")
    xg, wmap, pos, bexp, nblk = pl.kernel(
        _dispatch_body,
        out_type=(
            jax.ShapeDtypeStruct((NR, D // 2), jnp.int32),
            jax.ShapeDtypeStruct((NR,), jnp.float32),
            jax.ShapeDtypeStruct((P,), jnp.int32),
            jax.ShapeDtypeStruct((3 * L,), jnp.int32),
            jax.ShapeDtypeStruct((8,), jnp.int32),
        ),
        mesh=mesh,
        scratch_types=[
            pltpu.VMEM((PW,), jnp.int32),
            pltpu.VMEM((PW,), jnp.float32),
            pltpu.VMEM((PW,), jnp.int32),
            pltpu.VMEM((PW,), jnp.int32),
            pltpu.VMEM((PW // 2,), jnp.int32),
            pltpu.VMEM((PW // 2,), jnp.int32),
            pltpu.VMEM((L,), jnp.int32),
            pltpu.VMEM((NSUB * L,), jnp.int32),
            pltpu.VMEM((NR // NSUB,), jnp.int32),
            pltpu.VMEM((NR // NSUB,), jnp.float32),
            pltpu.VMEM((GR,), jnp.int32),
            pltpu.VMEM((GR,), jnp.float32),
            pltpu.VMEM((GC, D // 2), jnp.int32),
            pltpu.VMEM((GC, D // 2), jnp.int32),
            pltpu.VMEM((3 * L,), jnp.int32),
            pltpu.VMEM((L,), jnp.int32),
            pltpu.VMEM_SHARED((NSUB * L,), jnp.int32),
            pltpu.VMEM_SHARED((NR,), jnp.int32),
            pltpu.VMEM_SHARED((NR,), jnp.float32),
            pltpu.SemaphoreType.DMA,
            pltpu.SemaphoreType.DMA,
            pltpu.SemaphoreType.DMA,
            pltpu.SemaphoreType.DMA,
        ],
    )(sel.reshape(P), w.reshape(P), xp)

    bgu_g = bgu[:, 0::2].reshape(E, 1, F)
    bgu_u = bgu[:, 1::2].reshape(E, 1, F)

    grid_spec = pltpu.PrefetchScalarGridSpec(
        num_scalar_prefetch=2,
        grid=(NB,),
        in_specs=[
            pl.BlockSpec((BR, D // 2), lambda i, be, nb: (i, 0)),
            pl.BlockSpec((1, F, 2 * D), lambda i, be, nb: (be[i], 0, 0)),
            pl.BlockSpec((1, 1, F), lambda i, be, nb: (be[i], 0, 0)),
            pl.BlockSpec((1, 1, F), lambda i, be, nb: (be[i], 0, 0)),
            pl.BlockSpec((1, D, F), lambda i, be, nb: (be[i], 0, 0)),
            pl.BlockSpec((1, 1, D), lambda i, be, nb: (be[i], 0, 0)),
            pl.BlockSpec((1, BR, 1), lambda i, be, nb: (i, 0, 0)),
        ],
        out_specs=pl.BlockSpec((BR, D), lambda i, be, nb: (i, 0)),
    )
    yg = pl.pallas_call(
        _mlp_body,
        grid_spec=grid_spec,
        out_shape=jax.ShapeDtypeStruct((NR, D), jnp.float32),
    )(bexp, nblk, xg, Wgu.reshape(E, F, 2 * D),
      bgu_g, bgu_u, Wd, bd.reshape(E, 1, D), wmap.reshape(NB, BR, 1))

    out = pl.kernel(
        _combine_body,
        out_type=jax.ShapeDtypeStruct((S, D), jnp.float32),
        mesh=plsc.VectorSubcoreMesh(core_axis_name="c",
                                    subcore_axis_name="s"),
        scratch_types=[
            pltpu.VMEM((TPW * K,), jnp.int32),
            pltpu.VMEM((TPW, D), jnp.float32),
            pltpu.VMEM((TPW // 2, D), jnp.float32),
            pltpu.SemaphoreType.DMA,
        ],
    )(yg, pos)

    return out.reshape(B, S, D)


# speedup vs baseline: 1.8522x; 1.0431x over previous
import jax
import jax.numpy as jnp
from jax import lax
from jax.experimental import pallas as pl
from jax.experimental.pallas import tpu as pltpu
from jax.experimental.pallas import tpu_sc as plsc

B, S, D = 1, 2048, 768
E, K, F = 8, 2, 768
ALPHA, LIMIT = 1.702, 7.0

P = S * K
BR = 128
SH = 7
NB = P // BR + E
NR = NB * BR
L = 16
NC, NSUB = 2, 16
NW = NC * NSUB
PW = P // NSUB
VPW = PW // L
GR = NR // NW
GC = 80
TPW = S // NW
ZV = (NR // NSUB) // L


def _router_body(x_ref, wg_ref, bg_ref, sel_ref, w_ref, xp_ref):
    x = x_ref[...]
    logits = jax.lax.dot_general(
        x, wg_ref[...], (((1,), (1,)), ((), ())),
        preferred_element_type=jnp.float32)
    logits = logits + bg_ref[...]
    idx8 = jax.lax.broadcasted_iota(jnp.int32, (S, E), 1)
    m1 = jnp.max(logits, axis=1, keepdims=True)
    a1 = jnp.min(jnp.where(logits == m1, idx8, E), axis=1, keepdims=True)
    masked = jnp.where(idx8 == a1, -jnp.inf, logits)
    m2 = jnp.max(masked, axis=1, keepdims=True)
    a2 = jnp.min(jnp.where(masked == m2, idx8, E), axis=1, keepdims=True)
    w1 = jax.nn.sigmoid(m1 - m2)
    sel_ref[...] = jnp.concatenate([a1, a2], axis=1)
    w_ref[...] = jnp.concatenate([w1, 1.0 - w1], axis=1)
    xi = jax.lax.bitcast_convert_type(x, jnp.int32) + jnp.int32(0x8000)
    lo = (xi[:, :D // 2] >> 16) & jnp.int32(0xFFFF)
    hi = xi[:, D // 2:] & jnp.int32(-65536)
    xp_ref[...] = lo | hi


def _dispatch_body(sel_hbm, wp_hbm, x_hbm, xg_hbm, wmap_hbm, pos_hbm,
                   bexp_hbm, nblk_hbm, keyv, wv, rankv, tokv, posA, posB,
                   cnt_tbl, counts_all, zb_i, zb_f, idxv, wsl, rb0, rb1,
                   bexpv, nblkv, counts_sh, tok_sh, w_sh,
                   gs0, gs1, ws0, ws1):
    c = lax.axis_index("c")
    s = lax.axis_index("s")
    wid = s * NC + c
    pbase = s * PW
    iota = lax.iota(jnp.int32, L)
    zero16 = jnp.zeros((L,), jnp.int32)

    pltpu.sync_copy(sel_hbm.at[pl.ds(pbase, PW)], keyv)
    pltpu.sync_copy(wp_hbm.at[pl.ds(pbase, PW)], wv)
    run16 = zero16
    for j in range(VPW):
        k16 = keyv[pl.ds(j * L, L)]
        rank16 = zero16
        cnt16 = zero16
        for l in range(L):
            kl = k16[l]
            mask_l = jnp.where(iota > l, 1, 0)
            rank16 = rank16 + jnp.where(k16 == kl, 1, 0) * mask_l
            cnt16 = cnt16 + jnp.where(iota == kl, 1, 0)
        prev16 = zero16
        for e in range(E):
            prev16 = prev16 + jnp.where(
                k16 == e, jnp.full((L,), run16[e], jnp.int32), 0)
        rankv[pl.ds(j * L, L)] = prev16 + rank16
        run16 = run16 + cnt16
    cnt_tbl[...] = run16
    pltpu.sync_copy(cnt_tbl, counts_sh.at[pl.ds(s * L, L)])
    plsc.subcore_barrier()

    pltpu.sync_copy(counts_sh, counts_all)
    totals = zero16
    prefix = zero16
    for w2 in range(NSUB):
        c16 = counts_all[pl.ds(w2 * L, L)]
        m = -((w2 - s) >> 31)
        prefix = prefix + c16 * jnp.full((L,), m, jnp.int32)
        totals = totals + c16
    padded = ((totals + (BR - 1)) >> SH) << SH
    ends = zero16
    for e in range(E):
        ends = ends + jnp.where(
            iota >= e, jnp.full((L,), padded[e], jnp.int32), 0)
    offs = ends - padded
    base16 = offs + prefix
    base_sc = [base16[e] for e in range(E)]
    nblocks = ends[E - 1] >> SH

    for j in range(VPW):
        k16 = keyv[pl.ds(j * L, L)]
        b16 = zero16
        for e in range(E):
            b16 = b16 + jnp.where(
                k16 == e, jnp.full((L,), base_sc[e], jnp.int32), 0)
        p16 = b16 + rankv[pl.ds(j * L, L)]
        if j < VPW // 2:
            posA[pl.ds(j * L, L)] = p16
        else:
            posB[pl.ds((j - VPW // 2) * L, L)] = p16
        tokv[pl.ds(j * L, L)] = (
            jnp.full((L,), pbase + j * L, jnp.int32) + iota) >> 1

    @pl.when(c == 0)
    def _():
        pltpu.sync_copy(posA, pos_hbm.at[pl.ds(pbase, PW // 2)])
        pltpu.sync_copy(posB, pos_hbm.at[pl.ds(pbase + PW // 2, PW // 2)])

    zf16 = jnp.zeros((L,), jnp.float32)
    for j in range(ZV):
        zb_i[pl.ds(j * L, L)] = zero16
        zb_f[pl.ds(j * L, L)] = zf16
    pltpu.sync_copy(zb_i, tok_sh.at[pl.ds(s * (NR // NSUB), NR // NSUB)])
    pltpu.sync_copy(zb_f, w_sh.at[pl.ds(s * (NR // NSUB), NR // NSUB)])
    plsc.subcore_barrier()
    pltpu.sync_copy(tokv.at[pl.ds(0, PW // 2)], tok_sh.at[posA])
    pltpu.sync_copy(tokv.at[pl.ds(PW // 2, PW // 2)], tok_sh.at[posB])
    pltpu.sync_copy(wv.at[pl.ds(0, PW // 2)], w_sh.at[posA])
    pltpu.sync_copy(wv.at[pl.ds(PW // 2, PW // 2)], w_sh.at[posB])
    plsc.subcore_barrier()

    gbase = wid * GR
    pltpu.sync_copy(tok_sh.at[pl.ds(gbase, GR)], idxv)
    pltpu.sync_copy(w_sh.at[pl.ds(gbase, GR)], wsl)
    pltpu.sync_copy(wsl, wmap_hbm.at[pl.ds(gbase, GR)])
    g0 = pltpu.async_copy(x_hbm.at[idxv.at[pl.ds(0, GC)]], rb0, gs0)
    g0.wait()
    g1 = pltpu.async_copy(x_hbm.at[idxv.at[pl.ds(GC, GC)]], rb1, gs1)
    w0 = pltpu.async_copy(rb0, xg_hbm.at[pl.ds(gbase, GC)], ws0)
    g1.wait()
    w1 = pltpu.async_copy(rb1, xg_hbm.at[pl.ds(gbase + GC, GC)], ws1)
    w0.wait()
    w1.wait()


    @pl.when((c == 0) & (s == 0))
    def _():
        for v in range(3):
            bev = zero16
            stv = (iota + v * L) * BR
            for e in range(E):
                e16 = jnp.full((L,), ends[e], jnp.int32)
                bev = bev + jnp.where(stv >= e16, 1, 0)
            bexpv[pl.ds(v * L, L)] = jnp.minimum(bev, E - 1)
        pltpu.sync_copy(bexpv, bexp_hbm)
        nblkv[...] = jnp.full((L,), nblocks, jnp.int32)
        pltpu.sync_copy(nblkv.at[pl.ds(0, 8)], nblk_hbm)


def _mlp_body(bexp_ref, nblk_ref, xg_ref, wgu_ref, bgu_g_ref,
              bgu_u_ref, wd_ref, bd_ref, wrow_ref, yg_ref):
    i = pl.program_id(0)

    @pl.when(i < nblk_ref[0])
    def _():
        xgi = xg_ref[...]
        lo_f = jax.lax.bitcast_convert_type(xgi << 16, jnp.float32)
        hi_f = jax.lax.bitcast_convert_type(
            xgi & jnp.int32(-65536), jnp.float32)
        xb = jnp.concatenate([lo_f, hi_f], axis=1)
        wgu = wgu_ref[0]
        wg_ = wgu[:, :D]
        wu_ = wgu[:, D:]
        gate = jax.lax.dot_general(
            xb, wg_, (((1,), (1,)), ((), ())),
            preferred_element_type=jnp.float32) + bgu_g_ref[0]
        up = jax.lax.dot_general(
            xb, wu_, (((1,), (1,)), ((), ())),
            preferred_element_type=jnp.float32) + bgu_u_ref[0]
        gate = jnp.minimum(gate, LIMIT)
        up = jnp.clip(up, -LIMIT, LIMIT)
        glu = gate * jax.nn.sigmoid(gate * ALPHA)
        h = (up + 1.0) * glu
        y = jax.lax.dot_general(
            h, wd_ref[0], (((1,), (1,)), ((), ())),
            preferred_element_type=jnp.float32) + bd_ref[0]
        y = y * wrow_ref[0]
        yi = jax.lax.bitcast_convert_type(y, jnp.int32) + jnp.int32(0x8000)
        yg_ref[...] = ((yi[:, :D // 2] >> 16) & jnp.int32(0xFFFF)) | (
            yi[:, D // 2:] & jnp.int32(-65536))


def _combine_body(yg_hbm, pos_hbm, out_hbm, posv, buf, obuf, sem):
    c = lax.axis_index("c")
    s = lax.axis_index("s")
    wid = s * NC + c
    pltpu.sync_copy(pos_hbm.at[pl.ds(wid * TPW * K, TPW * K)], posv)
    for ch in range(2):
        pltpu.async_copy(
            yg_hbm.at[posv.at[pl.ds(ch * TPW, TPW)]], buf, sem).wait()

        def body_r(r, carry):
            for u in range(D // 2 // L):
                a = buf[2 * r, pl.ds(u * L, L)]
                b = buf[2 * r + 1, pl.ds(u * L, L)]
                alo = jax.lax.bitcast_convert_type(a << 16, jnp.float32)
                blo = jax.lax.bitcast_convert_type(b << 16, jnp.float32)
                ahi = jax.lax.bitcast_convert_type(
                    a & jnp.int32(-65536), jnp.float32)
                bhi = jax.lax.bitcast_convert_type(
                    b & jnp.int32(-65536), jnp.float32)
                obuf[r, pl.ds(u * L, L)] = alo + blo
                obuf[r, pl.ds(D // 2 + u * L, L)] = ahi + bhi
            return carry

        lax.fori_loop(0, TPW // 2, body_r, 0)
        pltpu.sync_copy(
            obuf, out_hbm.at[pl.ds(wid * TPW + ch * (TPW // 2), TPW // 2)])


def kernel(hidden_states, Wg, bg, Wgu, bgu, Wd, bd):
    x = hidden_states.reshape(S, D)

    sel, w, xp = pl.pallas_call(
        _router_body,
        out_shape=(
            jax.ShapeDtypeStruct((S, K), jnp.int32),
            jax.ShapeDtypeStruct((S, K), jnp.float32),
            jax.ShapeDtypeStruct((S, D // 2), jnp.int32),
        ),
    )(x, Wg, bg.reshape(1, E))

    mesh = plsc.VectorSubcoreMesh(core_axis_name="c", subcore_axis_name="s")
    xg, wmap, pos, bexp, nblk = pl.kernel(
        _dispatch_body,
        out_type=(
            jax.ShapeDtypeStruct((NR, D // 2), jnp.int32),
            jax.ShapeDtypeStruct((NR,), jnp.float32),
            jax.ShapeDtypeStruct((P,), jnp.int32),
            jax.ShapeDtypeStruct((3 * L,), jnp.int32),
            jax.ShapeDtypeStruct((8,), jnp.int32),
        ),
        mesh=mesh,
        scratch_types=[
            pltpu.VMEM((PW,), jnp.int32),
            pltpu.VMEM((PW,), jnp.float32),
            pltpu.VMEM((PW,), jnp.int32),
            pltpu.VMEM((PW,), jnp.int32),
            pltpu.VMEM((PW // 2,), jnp.int32),
            pltpu.VMEM((PW // 2,), jnp.int32),
            pltpu.VMEM((L,), jnp.int32),
            pltpu.VMEM((NSUB * L,), jnp.int32),
            pltpu.VMEM((NR // NSUB,), jnp.int32),
            pltpu.VMEM((NR // NSUB,), jnp.float32),
            pltpu.VMEM((GR,), jnp.int32),
            pltpu.VMEM((GR,), jnp.float32),
            pltpu.VMEM((GC, D // 2), jnp.int32),
            pltpu.VMEM((GC, D // 2), jnp.int32),
            pltpu.VMEM((3 * L,), jnp.int32),
            pltpu.VMEM((L,), jnp.int32),
            pltpu.VMEM_SHARED((NSUB * L,), jnp.int32),
            pltpu.VMEM_SHARED((NR,), jnp.int32),
            pltpu.VMEM_SHARED((NR,), jnp.float32),
            pltpu.SemaphoreType.DMA,
            pltpu.SemaphoreType.DMA,
            pltpu.SemaphoreType.DMA,
            pltpu.SemaphoreType.DMA,
        ],
    )(sel.reshape(P), w.reshape(P), xp)

    bgu_g = bgu[:, 0::2].reshape(E, 1, F)
    bgu_u = bgu[:, 1::2].reshape(E, 1, F)

    grid_spec = pltpu.PrefetchScalarGridSpec(
        num_scalar_prefetch=2,
        grid=(NB,),
        in_specs=[
            pl.BlockSpec((BR, D // 2), lambda i, be, nb: (i, 0)),
            pl.BlockSpec((1, F, 2 * D), lambda i, be, nb: (be[i], 0, 0)),
            pl.BlockSpec((1, 1, F), lambda i, be, nb: (be[i], 0, 0)),
            pl.BlockSpec((1, 1, F), lambda i, be, nb: (be[i], 0, 0)),
            pl.BlockSpec((1, D, F), lambda i, be, nb: (be[i], 0, 0)),
            pl.BlockSpec((1, 1, D), lambda i, be, nb: (be[i], 0, 0)),
            pl.BlockSpec((1, BR, 1), lambda i, be, nb: (i, 0, 0)),
        ],
        out_specs=pl.BlockSpec((BR, D // 2), lambda i, be, nb: (i, 0)),
    )
    yg = pl.pallas_call(
        _mlp_body,
        grid_spec=grid_spec,
        out_shape=jax.ShapeDtypeStruct((NR, D // 2), jnp.int32),
    )(bexp, nblk, xg, Wgu.reshape(E, F, 2 * D),
      bgu_g, bgu_u, Wd, bd.reshape(E, 1, D), wmap.reshape(NB, BR, 1))

    out = pl.kernel(
        _combine_body,
        out_type=jax.ShapeDtypeStruct((S, D), jnp.float32),
        mesh=plsc.VectorSubcoreMesh(core_axis_name="c",
                                    subcore_axis_name="s"),
        scratch_types=[
            pltpu.VMEM((TPW * K,), jnp.int32),
            pltpu.VMEM((TPW, D // 2), jnp.int32),
            pltpu.VMEM((TPW // 2, D), jnp.float32),
            pltpu.SemaphoreType.DMA,
        ],
    )(yg, pos)

    return out.reshape(B, S, D)
